# initial kernel scaffold (unmeasured)
import jax
import jax.numpy as jnp
from jax import lax
from jax.experimental import pallas as pl
from jax.experimental.pallas import tpu as pltpu

N_DEV = 8
N_TOK = 256
D_IN = 128
D_OUT = 256
N_EXP = 16
E_PER = N_EXP // N_DEV
CAP = 12
ROWS = N_TOK // N_DEV


def kernel(x, router_W, route_idx, expert_W):
    del router_W

    def body(x_ref, idx_ref, w_ref, out_ref, acc_ref, comm_ref, send_sems, recv_sems):
        my_i = lax.axis_index("i")

        e_col = idx_ref[:, :]
        eid = lax.broadcasted_iota(jnp.int32, (N_TOK, N_EXP), 1)
        onehot = (eid == e_col).astype(jnp.float32)
        row = lax.broadcasted_iota(jnp.int32, (N_TOK, N_TOK), 0)
        col = lax.broadcasted_iota(jnp.int32, (N_TOK, N_TOK), 1)
        lower = (row > col).astype(jnp.float32)
        pos16 = jnp.dot(lower, onehot, preferred_element_type=jnp.float32)
        pos = jnp.sum(pos16 * onehot, axis=1, keepdims=True)
        keep = (pos < CAP - 0.5).astype(jnp.float32)

        acc = jnp.zeros((N_TOK, D_OUT), dtype=jnp.float32)
        for l in range(E_PER):
            sel = (e_col == my_i * E_PER + l).astype(jnp.float32) * keep
            xm = x_ref[:, :] * sel
            acc = acc + jnp.dot(xm, w_ref[l], preferred_element_type=jnp.float32)
        acc_ref[:, :] = acc

        sends = []
        for r in range(1, N_DEV):
            dest = lax.rem(my_i + r, N_DEV)
            rdma = pltpu.make_async_remote_copy(
                src_ref=acc_ref.at[pl.ds(dest * ROWS, ROWS), :],
                dst_ref=comm_ref.at[N_DEV - r],
                send_sem=send_sems.at[N_DEV - r],
                recv_sem=recv_sems.at[N_DEV - r],
                device_id=(dest,),
                device_id_type=pl.DeviceIdType.MESH,
            )
            rdma.start()
            sends.append(rdma)

        total = acc_ref[pl.ds(my_i * ROWS, ROWS), :]
        for r in range(1, N_DEV):
            recv = pltpu.make_async_remote_copy(
                src_ref=comm_ref.at[r],
                dst_ref=comm_ref.at[r],
                send_sem=send_sems.at[r],
                recv_sem=recv_sems.at[r],
                device_id=(my_i,),
                device_id_type=pl.DeviceIdType.MESH,
            )
            recv.wait_recv()
            total = total + comm_ref[r]
        out_ref[:, :] = total

        for rdma in sends:
            rdma.wait_send()

    return pl.pallas_call(
        body,
        out_shape=jax.ShapeDtypeStruct((ROWS, D_OUT), jnp.float32),
        in_specs=[
            pl.BlockSpec(memory_space=pltpu.VMEM),
            pl.BlockSpec(memory_space=pltpu.VMEM),
            pl.BlockSpec(memory_space=pltpu.VMEM),
        ],
        out_specs=pl.BlockSpec(memory_space=pltpu.VMEM),
        scratch_shapes=[
            pltpu.VMEM((N_TOK, D_OUT), jnp.float32),
            pltpu.VMEM((N_DEV, ROWS, D_OUT), jnp.float32),
            pltpu.SemaphoreType.DMA((N_DEV,)),
            pltpu.SemaphoreType.DMA((N_DEV,)),
        ],
        compiler_params=pltpu.CompilerParams(collective_id=0),
    )(x, route_idx, expert_W)


# baseline (device time: 14315 ns/iter reference)
import jax
import jax.numpy as jnp
from jax import lax
from jax.experimental import pallas as pl
from jax.experimental.pallas import tpu as pltpu

N_DEV = 8
N_TOK = 256
D_IN = 128
D_OUT = 256
N_EXP = 16
E_PER = N_EXP // N_DEV
CAP = 12
ROWS = N_TOK // N_DEV


def kernel(x, router_W, route_idx, expert_W):
    del router_W

    def body(x_ref, idx_ref, w_ref, out_ref, acc_ref, comm_ref, send_sems, recv_sems):
        my_i = lax.axis_index("i")

        e_col = idx_ref[:, :]
        eid = lax.broadcasted_iota(jnp.int32, (N_TOK, N_EXP), 1)
        onehot = (eid == e_col).astype(jnp.float32)
        row = lax.broadcasted_iota(jnp.int32, (N_TOK, N_TOK), 0)
        col = lax.broadcasted_iota(jnp.int32, (N_TOK, N_TOK), 1)
        lower = (row > col).astype(jnp.float32)
        pos16 = jnp.dot(lower, onehot, preferred_element_type=jnp.float32)
        pos = jnp.sum(pos16 * onehot, axis=1, keepdims=True)
        keep = (pos < CAP - 0.5).astype(jnp.float32)

        acc = jnp.zeros((N_TOK, D_OUT), dtype=jnp.float32)
        for l in range(E_PER):
            sel = (e_col == my_i * E_PER + l).astype(jnp.float32) * keep
            xm = x_ref[:, :] * sel
            acc = acc + jnp.dot(xm, w_ref[l], preferred_element_type=jnp.float32)
        acc_ref[:, :] = acc

        sends = []
        for r in range(1, N_DEV):
            dest = lax.rem(my_i + r, N_DEV)
            rdma = pltpu.make_async_remote_copy(
                src_ref=acc_ref.at[pl.ds(dest * ROWS, ROWS), :],
                dst_ref=comm_ref.at[N_DEV - r],
                send_sem=send_sems.at[N_DEV - r],
                recv_sem=recv_sems.at[N_DEV - r],
                device_id=(dest,),
                device_id_type=pl.DeviceIdType.MESH,
            )
            rdma.start()
            sends.append(rdma)

        total = acc_ref[pl.ds(my_i * ROWS, ROWS), :]
        for r in range(1, N_DEV):
            recv = pltpu.make_async_remote_copy(
                src_ref=comm_ref.at[r],
                dst_ref=comm_ref.at[r],
                send_sem=send_sems.at[r],
                recv_sem=recv_sems.at[r],
                device_id=(my_i,),
                device_id_type=pl.DeviceIdType.MESH,
            )
            recv.wait_recv()
            total = total + comm_ref[r]
        out_ref[:, :] = total

        for rdma in sends:
            rdma.wait_send()

    return pl.pallas_call(
        body,
        out_shape=jax.ShapeDtypeStruct((ROWS, D_OUT), jnp.float32),
        in_specs=[
            pl.BlockSpec(memory_space=pltpu.VMEM),
            pl.BlockSpec(memory_space=pltpu.VMEM),
            pl.BlockSpec(memory_space=pltpu.VMEM),
        ],
        out_specs=pl.BlockSpec(memory_space=pltpu.VMEM),
        scratch_shapes=[
            pltpu.VMEM((N_TOK, D_OUT), jnp.float32),
            pltpu.VMEM((N_DEV, ROWS, D_OUT), jnp.float32),
            pltpu.SemaphoreType.DMA((N_DEV,)),
            pltpu.SemaphoreType.DMA((N_DEV,)),
        ],
    )(x, route_idx, expert_W)
